# baseline (speedup 1.0000x reference)
import functools

import jax
import jax.numpy as jnp
from jax import lax
from jax.experimental import pallas as pl
from jax.experimental.pallas import tpu as pltpu
from jax.experimental.pallas import tpu_sc as plsc

N = 10000
E = 320000
D = 128
HH = 128
RW = 256
NADD = 8
GW = 16
XW = 16

NC = 2
NS = 16
NW = NC * NS
CH = 128

NCHUNK = 2
ECH = E // NCHUNK
NPART = NC * NCHUNK

_HI = lax.Precision.HIGHEST
_F32 = jnp.float32


def _psi(v):
    return jnp.sign(v) * jnp.log1p(jnp.abs(v))


_BN_PREP = 2000


def _prep_body(h_ref, x_ref, w1a_ref, w1b_ref, rp_ref, rq_ref):
    hb = h_ref[...]
    xb = x_ref[...]
    p = jnp.dot(hb, w1a_ref[...], preferred_element_type=_F32,
                precision=_HI)
    q = jnp.dot(hb, w1b_ref[...], preferred_element_type=_F32,
                precision=_HI)
    n2 = (xb[:, 0:1] ** 2 - xb[:, 1:2] ** 2 - xb[:, 2:3] ** 2
          - xb[:, 3:4] ** 2)
    pad = jnp.zeros((xb.shape[0], RW - HH - 5), _F32)
    rp_ref[...] = jnp.concatenate([p, xb, n2, pad], axis=1)
    rq_ref[...] = jnp.concatenate([q, -xb, n2, pad], axis=1)


def _prep(h, x, w1a, w1b):
    nblk = N // _BN_PREP
    return pl.pallas_call(
        _prep_body,
        grid=(nblk,),
        in_specs=[
            pl.BlockSpec((_BN_PREP, D), lambda n: (n, 0)),
            pl.BlockSpec((_BN_PREP, 4), lambda n: (n, 0)),
            pl.BlockSpec((D, HH), lambda n: (0, 0)),
            pl.BlockSpec((D, HH), lambda n: (0, 0)),
        ],
        out_specs=[
            pl.BlockSpec((_BN_PREP, RW), lambda n: (n, 0)),
            pl.BlockSpec((_BN_PREP, RW), lambda n: (n, 0)),
        ],
        out_shape=[
            jax.ShapeDtypeStruct((N, RW), _F32),
            jax.ShapeDtypeStruct((N, RW), _F32),
        ],
    )(h, x, w1a, w1b)


def _sc_gather(rptab, rqtab, ivec, jvec):
    ne = ivec.shape[0]
    epw = ne // NW
    nfull = epw // CH
    tail = epw - nfull * CH
    mesh = plsc.VectorSubcoreMesh(core_axis_name="c", subcore_axis_name="s",
                                  num_cores=NC, num_subcores=NS)

    @functools.partial(
        pl.kernel,
        out_type=[
            jax.ShapeDtypeStruct((ne, HH), _F32),
            jax.ShapeDtypeStruct((ne, GW), _F32),
        ],
        mesh=mesh,
        scratch_types=[
            pltpu.VMEM((CH,), jnp.int32),
            pltpu.VMEM((CH,), jnp.int32),
            pltpu.VMEM((CH, RW), _F32),
            pltpu.VMEM((CH, RW), _F32),
            pltpu.VMEM((CH, GW), _F32),
            pltpu.VMEM((max(tail, 1),), jnp.int32),
            pltpu.VMEM((max(tail, 1),), jnp.int32),
            pltpu.VMEM((max(tail, 1), RW), _F32),
            pltpu.VMEM((max(tail, 1), RW), _F32),
            pltpu.VMEM((max(tail, 1), GW), _F32),
            pltpu.SemaphoreType.DMA,
            pltpu.SemaphoreType.DMA,
        ],
    )
    def gather_kernel(rp_hbm, rq_hbm, i_hbm, j_hbm, s_hbm, g_hbm,
                      iv, jv, bp, bq, bg, ivt, jvt, bpt, bqt, bgt,
                      semp, semq):
        wid = lax.axis_index("s") * NC + lax.axis_index("c")
        base = wid * epw
        gsl = pl.ds(HH, 16)

        @pl.loop(0, nfull)
        def _chunk(k):
            off = base + k * CH
            pltpu.sync_copy(i_hbm.at[pl.ds(off, CH)], iv)
            pltpu.sync_copy(j_hbm.at[pl.ds(off, CH)], jv)
            cp = pltpu.async_copy(rp_hbm.at[iv], bp, semp)
            cq = pltpu.async_copy(rq_hbm.at[jv], bq, semq)
            cp.wait()
            cq.wait()

            @pl.loop(0, CH)
            def _row(r):
                for c in range(NADD):
                    sl = pl.ds(c * 16, 16)
                    bp[r, sl] = bp[r, sl] + bq[r, sl]
                bg[r, pl.ds(0, 16)] = bp[r, gsl] + bq[r, gsl]

            pltpu.sync_copy(bp.at[:, pl.ds(0, HH)], s_hbm.at[pl.ds(off, CH)])
            pltpu.sync_copy(bg, g_hbm.at[pl.ds(off, CH)])

        if tail:
            offt = base + nfull * CH
            pltpu.sync_copy(i_hbm.at[pl.ds(offt, tail)], ivt)
            pltpu.sync_copy(j_hbm.at[pl.ds(offt, tail)], jvt)
            cp = pltpu.async_copy(rp_hbm.at[ivt], bpt, semp)
            cq = pltpu.async_copy(rq_hbm.at[jvt], bqt, semq)
            cp.wait()
            cq.wait()

            @pl.loop(0, tail)
            def _rowt(r):
                for c in range(NADD):
                    sl = pl.ds(c * 16, 16)
                    bpt[r, sl] = bpt[r, sl] + bqt[r, sl]
                bgt[r, pl.ds(0, 16)] = bpt[r, gsl] + bqt[r, gsl]

            pltpu.sync_copy(bpt.at[:, pl.ds(0, HH)],
                            s_hbm.at[pl.ds(offt, tail)])
            pltpu.sync_copy(bgt, g_hbm.at[pl.ds(offt, tail)])

    return gather_kernel(rptab, rqtab, ivec, jvec)


def _edge_out(s0, g, wn, wd):
    xd = g[:, 0:4]
    n2s = g[:, 4:5]
    norms = (xd[:, 0:1] ** 2 - xd[:, 1:2] ** 2 - xd[:, 2:3] ** 2
             - xd[:, 3:4] ** 2)
    dots = 0.5 * (n2s - norms)
    out = s0 + _psi(norms) * wn + _psi(dots) * wd
    return out, xd


_BE1 = 3200


def _stats_body(s_ref, g_ref, wn_ref, wd_ref, p_ref, acc_ref):
    pid = pl.program_id(0)

    @pl.when(pid == 0)
    def _():
        acc_ref[...] = jnp.zeros_like(acc_ref)

    out, _ = _edge_out(s_ref[...], g_ref[...], wn_ref[...], wd_ref[...])
    acc_ref[0:1, :] += jnp.sum(out, axis=0, keepdims=True)
    acc_ref[1:2, :] += jnp.sum(out * out, axis=0, keepdims=True)

    @pl.when(pid == pl.num_programs(0) - 1)
    def _():
        p_ref[...] = acc_ref[...]


def _edge_stats(s, g, wn, wd):
    ne = s.shape[0]
    nblk = ne // _BE1
    return pl.pallas_call(
        _stats_body,
        grid=(nblk,),
        in_specs=[
            pl.BlockSpec((_BE1, HH), lambda e: (e, 0)),
            pl.BlockSpec((_BE1, GW), lambda e: (e, 0)),
            pl.BlockSpec((1, HH), lambda e: (0, 0)),
            pl.BlockSpec((1, HH), lambda e: (0, 0)),
        ],
        out_specs=pl.BlockSpec((2, HH), lambda e: (0, 0)),
        out_shape=jax.ShapeDtypeStruct((2, HH), _F32),
        scratch_shapes=[pltpu.VMEM((2, HH), _F32)],
    )(s, g, wn, wd)


def _comb_body(ge_ref, be_ref, ab_ref, *p_refs):
    s = p_refs[0][...]
    for p in p_refs[1:]:
        s = s + p[...]
    mu = s[0:1, :] * (1.0 / E)
    var = s[1:2, :] * (1.0 / E) - mu * mu
    a = ge_ref[...] * lax.rsqrt(var + 1e-5)
    b = be_ref[...] - mu * a
    ab_ref[...] = jnp.concatenate([a, b], axis=0)


def _stats_combine(parts, ge, be):
    def body(*refs):
        ge_ref, be_ref = refs[len(parts)], refs[len(parts) + 1]
        ab_ref = refs[len(parts) + 2]
        _comb_body(ge_ref, be_ref, ab_ref, *refs[:len(parts)])

    return pl.pallas_call(
        body,
        grid=(1,),
        in_specs=[pl.BlockSpec((2, HH), lambda e: (0, 0)) for _ in parts]
        + [pl.BlockSpec((1, HH), lambda e: (0, 0)),
           pl.BlockSpec((1, HH), lambda e: (0, 0))],
        out_specs=pl.BlockSpec((2, HH), lambda e: (0, 0)),
        out_shape=jax.ShapeDtypeStruct((2, HH), _F32),
    )(*parts, ge, be)


_BE2 = 2000


def _mlp_body(s_ref, g_ref, wn_ref, wd_ref, ab_ref,
              we2_ref, be2_ref, wmt_ref, bm_ref, wx1_ref, bx1_ref, wx2t_ref,
              m_ref, t16_ref):
    out, xd = _edge_out(s_ref[...], g_ref[...], wn_ref[...], wd_ref[...])
    out = jnp.maximum(out * ab_ref[0:1, :] + ab_ref[1:2, :], 0.0)
    out2 = jnp.maximum(
        jnp.dot(out, we2_ref[...], preferred_element_type=_F32,
                precision=_HI) + be2_ref[...], 0.0)
    logit = jnp.sum(out2 * wmt_ref[...], axis=1, keepdims=True) + bm_ref[0, 0]
    w = jax.nn.sigmoid(logit)
    m = out2 * w
    m_ref[...] = m
    t = jnp.maximum(
        jnp.dot(m, wx1_ref[...], preferred_element_type=_F32,
                precision=_HI) + bx1_ref[...], 0.0)
    xm = jnp.sum(t * wx2t_ref[...], axis=1, keepdims=True)
    trans = jnp.clip(xd * xm, -100.0, 100.0)
    nrow = trans.shape[0]
    t16_ref[...] = jnp.concatenate(
        [trans, jnp.ones((nrow, 1), _F32), jnp.zeros((nrow, XW - 5), _F32)],
        axis=1)


def _edge_mlp(s, g, wn, wd, ab, we2, be2, wmt, bm, wx1, bx1, wx2t):
    ne = s.shape[0]
    nblk = ne // _BE2
    return pl.pallas_call(
        _mlp_body,
        grid=(nblk,),
        in_specs=[
            pl.BlockSpec((_BE2, HH), lambda e: (e, 0)),
            pl.BlockSpec((_BE2, GW), lambda e: (e, 0)),
            pl.BlockSpec((1, HH), lambda e: (0, 0)),
            pl.BlockSpec((1, HH), lambda e: (0, 0)),
            pl.BlockSpec((2, HH), lambda e: (0, 0)),
            pl.BlockSpec((HH, HH), lambda e: (0, 0)),
            pl.BlockSpec((1, HH), lambda e: (0, 0)),
            pl.BlockSpec((1, HH), lambda e: (0, 0)),
            pl.BlockSpec((1, 1), lambda e: (0, 0)),
            pl.BlockSpec((HH, HH), lambda e: (0, 0)),
            pl.BlockSpec((1, HH), lambda e: (0, 0)),
            pl.BlockSpec((1, HH), lambda e: (0, 0)),
        ],
        out_specs=[
            pl.BlockSpec((_BE2, HH), lambda e: (e, 0)),
            pl.BlockSpec((_BE2, XW), lambda e: (e, 0)),
        ],
        out_shape=[
            jax.ShapeDtypeStruct((ne, HH), _F32),
            jax.ShapeDtypeStruct((ne, XW), _F32),
        ],
    )(s, g, wn, wd, ab, we2, be2, wmt, bm, wx1, bx1, wx2t)


_NSTRIPE = N // 16
_NROUND = _NSTRIPE // NS
_NXTRA = _NSTRIPE - _NROUND * NS


def _sc_scatter(src_width, ne):
    epw = ne // NW
    nfull = epw // CH
    tail = epw - nfull * CH
    inflate = src_width != HH
    mesh = plsc.VectorSubcoreMesh(core_axis_name="c", subcore_axis_name="s",
                                  num_cores=NC, num_subcores=NS)

    scratch = [
        pltpu.MemorySpace.VMEM_SHARED((N, HH), _F32),
        pltpu.VMEM((CH,), jnp.int32),
        pltpu.VMEM((CH, src_width), _F32),
        pltpu.VMEM((max(tail, 1),), jnp.int32),
        pltpu.VMEM((max(tail, 1), src_width), _F32),
    ]
    if inflate:
        scratch += [pltpu.VMEM((CH, HH), _F32),
                    pltpu.VMEM((max(tail, 1), HH), _F32)]

    @functools.partial(
        pl.kernel,
        out_type=jax.ShapeDtypeStruct((NC, N, HH), _F32),
        mesh=mesh,
        scratch_types=scratch,
    )
    def scatter_kernel(rows_hbm, i_hbm, acc_hbm, acc_sh, iv, rb, ivt, rbt,
                       *inf):
        cid = lax.axis_index("c")
        sid = lax.axis_index("s")
        wid = sid * NC + cid
        zv = jnp.zeros((16,), _F32)
        zbuf = inf[0] if inflate else rb

        @pl.loop(0, 16)
        def _zm(r):
            for c in range(HH // 16):
                zbuf[r, pl.ds(c * 16, 16)] = zv

        @pl.loop(0, _NROUND)
        def _clear(k):
            off = (k * NS + sid) * 16
            pltpu.sync_copy(zbuf.at[pl.ds(0, 16)], acc_sh.at[pl.ds(off, 16)])

        @pl.when(sid < _NXTRA)
        def _():
            off = (_NROUND * NS + sid) * 16
            pltpu.sync_copy(zbuf.at[pl.ds(0, 16)], acc_sh.at[pl.ds(off, 16)])

        if inflate:
            @pl.loop(16, CH)
            def _zrest(r):
                for c in range(HH // 16):
                    inf[0][r, pl.ds(c * 16, 16)] = zv

            @pl.loop(0, tail)
            def _zrt(r):
                for c in range(HH // 16):
                    inf[1][r, pl.ds(c * 16, 16)] = zv

        plsc.subcore_barrier()

        base = wid * epw

        def _do_chunk(off, n, ivb, rbb, ibuf):
            pltpu.sync_copy(i_hbm.at[pl.ds(off, n)], ivb)
            pltpu.sync_copy(rows_hbm.at[pl.ds(off, n)], rbb)
            if inflate:
                @pl.loop(0, n)
                def _inf(r):
                    ibuf[r, pl.ds(0, src_width)] = rbb[r, pl.ds(0, src_width)]

                pltpu.sync_copy(ibuf, acc_sh.at[ivb], add=True)
            else:
                pltpu.sync_copy(rbb, acc_sh.at[ivb], add=True)

        @pl.loop(0, nfull)
        def _chunk(k):
            _do_chunk(base + k * CH, CH, iv, rb, inf[0] if inflate else None)

        if tail:
            _do_chunk(base + nfull * CH, tail, ivt, rbt,
                      inf[1] if inflate else None)

        plsc.subcore_barrier()

        @pl.loop(0, _NROUND)
        def _flush(k):
            off = (k * NS + sid) * 16
            pltpu.sync_copy(acc_sh.at[pl.ds(off, 16)],
                            acc_hbm.at[cid, pl.ds(off, 16)])

        @pl.when(sid < _NXTRA)
        def _():
            off = (_NROUND * NS + sid) * 16
            pltpu.sync_copy(acc_sh.at[pl.ds(off, 16)],
                            acc_hbm.at[cid, pl.ds(off, 16)])

    return scatter_kernel


_BN_NODE = 2000


def _node1_body(h_ref, aggp_ref, na_ref, w1_ref, w2_ref, wc_ref, bh1_ref,
                gh_ref, bh_ref, hh_ref, ab2_ref, acc_ref):
    pid = pl.program_id(0)

    @pl.when(pid == 0)
    def _():
        acc_ref[...] = jnp.zeros_like(acc_ref)

    agg = aggp_ref[0]
    for k in range(1, NPART):
        agg = agg + aggp_ref[k]
    na = na_ref[...]
    hh = (jnp.dot(h_ref[...], w1_ref[...], preferred_element_type=_F32,
                  precision=_HI)
          + jnp.dot(agg, w2_ref[...], preferred_element_type=_F32,
                    precision=_HI)
          + na[:, 0:1] * wc_ref[0:1, :] + na[:, 1:2] * wc_ref[1:2, :]
          + bh1_ref[...])
    hh_ref[...] = hh
    acc_ref[0:1, :] += jnp.sum(hh, axis=0, keepdims=True)
    acc_ref[1:2, :] += jnp.sum(hh * hh, axis=0, keepdims=True)

    @pl.when(pid == pl.num_programs(0) - 1)
    def _():
        mu = acc_ref[0:1, :] * (1.0 / N)
        var = acc_ref[1:2, :] * (1.0 / N) - mu * mu
        a = gh_ref[...] * lax.rsqrt(var + 1e-5)
        b = bh_ref[...] - mu * a
        ab2_ref[...] = jnp.concatenate([a, b], axis=0)


def _node1(h, aggp, node_attr, w1, w2, wc, bh1, gh, bh):
    nblk = N // _BN_NODE
    return pl.pallas_call(
        _node1_body,
        grid=(nblk,),
        in_specs=[
            pl.BlockSpec((_BN_NODE, D), lambda n: (n, 0)),
            pl.BlockSpec((NPART, _BN_NODE, HH), lambda n: (0, n, 0)),
            pl.BlockSpec((_BN_NODE, 2), lambda n: (n, 0)),
            pl.BlockSpec((D, HH), lambda n: (0, 0)),
            pl.BlockSpec((HH, HH), lambda n: (0, 0)),
            pl.BlockSpec((2, HH), lambda n: (0, 0)),
            pl.BlockSpec((1, HH), lambda n: (0, 0)),
            pl.BlockSpec((1, HH), lambda n: (0, 0)),
            pl.BlockSpec((1, HH), lambda n: (0, 0)),
        ],
        out_specs=[
            pl.BlockSpec((_BN_NODE, HH), lambda n: (n, 0)),
            pl.BlockSpec((2, HH), lambda n: (0, 0)),
        ],
        out_shape=[
            jax.ShapeDtypeStruct((N, HH), _F32),
            jax.ShapeDtypeStruct((2, HH), _F32),
        ],
        scratch_shapes=[pltpu.VMEM((2, HH), _F32)],
    )(h, aggp, node_attr, w1, w2, wc, bh1, gh, bh)


def _node2_body(hh_ref, ab2_ref, h_ref, wh2_ref, bh2_ref, x_ref, tsp_ref,
                hn_ref, xn_ref):
    hh = jnp.maximum(hh_ref[...] * ab2_ref[0:1, :] + ab2_ref[1:2, :], 0.0)
    hn_ref[...] = (h_ref[...]
                   + jnp.dot(hh, wh2_ref[...], preferred_element_type=_F32,
                             precision=_HI) + bh2_ref[...])
    ts = tsp_ref[0]
    for k in range(1, NPART):
        ts = ts + tsp_ref[k]
    cnt = jnp.maximum(ts[:, 4:5], 1.0)
    xn_ref[...] = x_ref[...] + ts[:, 0:4] / cnt


def _node2(hh, ab2, h, wh2, bh2, x, tsp):
    nblk = N // _BN_NODE
    return pl.pallas_call(
        _node2_body,
        grid=(nblk,),
        in_specs=[
            pl.BlockSpec((_BN_NODE, HH), lambda n: (n, 0)),
            pl.BlockSpec((2, HH), lambda n: (0, 0)),
            pl.BlockSpec((_BN_NODE, D), lambda n: (n, 0)),
            pl.BlockSpec((HH, D), lambda n: (0, 0)),
            pl.BlockSpec((1, D), lambda n: (0, 0)),
            pl.BlockSpec((_BN_NODE, 4), lambda n: (n, 0)),
            pl.BlockSpec((NPART, _BN_NODE, HH), lambda n: (0, n, 0)),
        ],
        out_specs=[
            pl.BlockSpec((_BN_NODE, D), lambda n: (n, 0)),
            pl.BlockSpec((_BN_NODE, 4), lambda n: (n, 0)),
        ],
        out_shape=[
            jax.ShapeDtypeStruct((N, D), _F32),
            jax.ShapeDtypeStruct((N, 4), _F32),
        ],
    )(hh, ab2, h, wh2, bh2, x, tsp)


def kernel(h, x, edges, node_attr, we1, ge, be, we2, be2, wm, bm, wh1, bh1,
           gh, bh, wh2, bh2, wx1, bx1, wx2):
    ivec = edges[0]
    jvec = edges[1]
    w1a = we1[:D]
    w1b = we1[D:2 * D]
    wn = we1[2 * D].reshape(1, HH)
    wd = we1[2 * D + 1].reshape(1, HH)

    rptab, rqtab = _prep(h, x, w1a, w1b)

    ivs = [lax.slice_in_dim(ivec, k * ECH, (k + 1) * ECH)
           for k in range(NCHUNK)]
    jvs = [lax.slice_in_dim(jvec, k * ECH, (k + 1) * ECH)
           for k in range(NCHUNK)]

    ss = [_sc_gather(rptab, rqtab, ivs[k], jvs[k]) for k in range(NCHUNK)]
    parts = [_edge_stats(ss[k][0], ss[k][1], wn, wd) for k in range(NCHUNK)]
    ab = _stats_combine(parts, ge.reshape(1, HH), be.reshape(1, HH))

    ms, t16s = [], []
    for k in range(NCHUNK):
        mk, tk = _edge_mlp(ss[k][0], ss[k][1], wn, wd, ab, we2,
                           be2.reshape(1, HH), wm.reshape(1, HH),
                           bm.reshape(1, 1), wx1, bx1.reshape(1, HH),
                           wx2.reshape(1, HH))
        ms.append(mk)
        t16s.append(tk)

    aggps = [_sc_scatter(HH, ECH)(ms[k], ivs[k]) for k in range(NCHUNK)]
    tsps = [_sc_scatter(XW, ECH)(t16s[k], ivs[k]) for k in range(NCHUNK)]

    aggp = jnp.concatenate(aggps, axis=0)
    tsp = jnp.concatenate(tsps, axis=0)
    m = jnp.concatenate(ms, axis=0)

    hhpre, ab2 = _node1(h, aggp, node_attr, wh1[:D], wh1[D:2 * D],
                        wh1[2 * D:], bh1.reshape(1, HH), gh.reshape(1, HH),
                        bh.reshape(1, HH))
    h_new, x_new = _node2(hhpre, ab2, h, wh2, bh2.reshape(1, D), x, tsp)
    return (h_new, x_new, m)


# speedup vs baseline: 1.2553x; 1.2553x over previous
import functools

import jax
import jax.numpy as jnp
from jax import lax
from jax.experimental import pallas as pl
from jax.experimental.pallas import tpu as pltpu
from jax.experimental.pallas import tpu_sc as plsc

N = 10000
E = 320000
D = 128
HH = 128
RW = 256
NADD = 8
GW = 16
XW = 16

NC = 2
NS = 16
NW = NC * NS
CH = 128

NCHUNK = 2
ECH = E // NCHUNK
NPART = NC * NCHUNK

_HI = lax.Precision.DEFAULT
_F32 = jnp.float32


def _psi(v):
    return jnp.sign(v) * jnp.log1p(jnp.abs(v))


_BN_PREP = 2000


def _prep_body(h_ref, x_ref, w1a_ref, w1b_ref, rp_ref, rq_ref):
    hb = h_ref[...]
    xb = x_ref[...]
    p = jnp.dot(hb, w1a_ref[...], preferred_element_type=_F32,
                precision=_HI)
    q = jnp.dot(hb, w1b_ref[...], preferred_element_type=_F32,
                precision=_HI)
    n2 = (xb[:, 0:1] ** 2 - xb[:, 1:2] ** 2 - xb[:, 2:3] ** 2
          - xb[:, 3:4] ** 2)
    pad = jnp.zeros((xb.shape[0], RW - HH - 5), _F32)
    rp_ref[...] = jnp.concatenate([p, xb, n2, pad], axis=1)
    rq_ref[...] = jnp.concatenate([q, -xb, n2, pad], axis=1)


def _prep(h, x, w1a, w1b):
    nblk = N // _BN_PREP
    return pl.pallas_call(
        _prep_body,
        grid=(nblk,),
        in_specs=[
            pl.BlockSpec((_BN_PREP, D), lambda n: (n, 0)),
            pl.BlockSpec((_BN_PREP, 4), lambda n: (n, 0)),
            pl.BlockSpec((D, HH), lambda n: (0, 0)),
            pl.BlockSpec((D, HH), lambda n: (0, 0)),
        ],
        out_specs=[
            pl.BlockSpec((_BN_PREP, RW), lambda n: (n, 0)),
            pl.BlockSpec((_BN_PREP, RW), lambda n: (n, 0)),
        ],
        out_shape=[
            jax.ShapeDtypeStruct((N, RW), _F32),
            jax.ShapeDtypeStruct((N, RW), _F32),
        ],
    )(h, x, w1a, w1b)


def _sc_gather(rptab, rqtab, ivec, jvec):
    ne = ivec.shape[0]
    epw = ne // NW
    nfull = epw // CH
    tail = epw - nfull * CH
    mesh = plsc.VectorSubcoreMesh(core_axis_name="c", subcore_axis_name="s",
                                  num_cores=NC, num_subcores=NS)

    @functools.partial(
        pl.kernel,
        out_type=[
            jax.ShapeDtypeStruct((ne, HH), _F32),
            jax.ShapeDtypeStruct((ne, GW), _F32),
        ],
        mesh=mesh,
        scratch_types=[
            pltpu.VMEM((CH,), jnp.int32),
            pltpu.VMEM((CH,), jnp.int32),
            pltpu.VMEM((CH, RW), _F32),
            pltpu.VMEM((CH, RW), _F32),
            pltpu.VMEM((CH, GW), _F32),
            pltpu.VMEM((max(tail, 1),), jnp.int32),
            pltpu.VMEM((max(tail, 1),), jnp.int32),
            pltpu.VMEM((max(tail, 1), RW), _F32),
            pltpu.VMEM((max(tail, 1), RW), _F32),
            pltpu.VMEM((max(tail, 1), GW), _F32),
            pltpu.SemaphoreType.DMA,
            pltpu.SemaphoreType.DMA,
        ],
    )
    def gather_kernel(rp_hbm, rq_hbm, i_hbm, j_hbm, s_hbm, g_hbm,
                      iv, jv, bp, bq, bg, ivt, jvt, bpt, bqt, bgt,
                      semp, semq):
        wid = lax.axis_index("s") * NC + lax.axis_index("c")
        base = wid * epw
        gsl = pl.ds(HH, 16)

        @pl.loop(0, nfull)
        def _chunk(k):
            off = base + k * CH
            pltpu.sync_copy(i_hbm.at[pl.ds(off, CH)], iv)
            pltpu.sync_copy(j_hbm.at[pl.ds(off, CH)], jv)
            cp = pltpu.async_copy(rp_hbm.at[iv], bp, semp)
            cq = pltpu.async_copy(rq_hbm.at[jv], bq, semq)
            cp.wait()
            cq.wait()

            @pl.loop(0, CH)
            def _row(r):
                for c in range(NADD):
                    sl = pl.ds(c * 16, 16)
                    bp[r, sl] = bp[r, sl] + bq[r, sl]
                bg[r, pl.ds(0, 16)] = bp[r, gsl] + bq[r, gsl]

            pltpu.sync_copy(bp.at[:, pl.ds(0, HH)], s_hbm.at[pl.ds(off, CH)])
            pltpu.sync_copy(bg, g_hbm.at[pl.ds(off, CH)])

        if tail:
            offt = base + nfull * CH
            pltpu.sync_copy(i_hbm.at[pl.ds(offt, tail)], ivt)
            pltpu.sync_copy(j_hbm.at[pl.ds(offt, tail)], jvt)
            cp = pltpu.async_copy(rp_hbm.at[ivt], bpt, semp)
            cq = pltpu.async_copy(rq_hbm.at[jvt], bqt, semq)
            cp.wait()
            cq.wait()

            @pl.loop(0, tail)
            def _rowt(r):
                for c in range(NADD):
                    sl = pl.ds(c * 16, 16)
                    bpt[r, sl] = bpt[r, sl] + bqt[r, sl]
                bgt[r, pl.ds(0, 16)] = bpt[r, gsl] + bqt[r, gsl]

            pltpu.sync_copy(bpt.at[:, pl.ds(0, HH)],
                            s_hbm.at[pl.ds(offt, tail)])
            pltpu.sync_copy(bgt, g_hbm.at[pl.ds(offt, tail)])

    return gather_kernel(rptab, rqtab, ivec, jvec)


def _edge_out(s0, g, wn, wd):
    xd = g[:, 0:4]
    n2s = g[:, 4:5]
    norms = (xd[:, 0:1] ** 2 - xd[:, 1:2] ** 2 - xd[:, 2:3] ** 2
             - xd[:, 3:4] ** 2)
    dots = 0.5 * (n2s - norms)
    out = s0 + _psi(norms) * wn + _psi(dots) * wd
    return out, xd


_BE1 = 3200


def _stats_body(s_ref, g_ref, wn_ref, wd_ref, p_ref, acc_ref):
    pid = pl.program_id(0)

    @pl.when(pid == 0)
    def _():
        acc_ref[...] = jnp.zeros_like(acc_ref)

    out, _ = _edge_out(s_ref[...], g_ref[...], wn_ref[...], wd_ref[...])
    acc_ref[0:1, :] += jnp.sum(out, axis=0, keepdims=True)
    acc_ref[1:2, :] += jnp.sum(out * out, axis=0, keepdims=True)

    @pl.when(pid == pl.num_programs(0) - 1)
    def _():
        p_ref[...] = acc_ref[...]


def _edge_stats(s, g, wn, wd):
    ne = s.shape[0]
    nblk = ne // _BE1
    return pl.pallas_call(
        _stats_body,
        grid=(nblk,),
        in_specs=[
            pl.BlockSpec((_BE1, HH), lambda e: (e, 0)),
            pl.BlockSpec((_BE1, GW), lambda e: (e, 0)),
            pl.BlockSpec((1, HH), lambda e: (0, 0)),
            pl.BlockSpec((1, HH), lambda e: (0, 0)),
        ],
        out_specs=pl.BlockSpec((2, HH), lambda e: (0, 0)),
        out_shape=jax.ShapeDtypeStruct((2, HH), _F32),
        scratch_shapes=[pltpu.VMEM((2, HH), _F32)],
    )(s, g, wn, wd)


def _comb_body(ge_ref, be_ref, ab_ref, *p_refs):
    s = p_refs[0][...]
    for p in p_refs[1:]:
        s = s + p[...]
    mu = s[0:1, :] * (1.0 / E)
    var = s[1:2, :] * (1.0 / E) - mu * mu
    a = ge_ref[...] * lax.rsqrt(var + 1e-5)
    b = be_ref[...] - mu * a
    ab_ref[...] = jnp.concatenate([a, b], axis=0)


def _stats_combine(parts, ge, be):
    def body(*refs):
        ge_ref, be_ref = refs[len(parts)], refs[len(parts) + 1]
        ab_ref = refs[len(parts) + 2]
        _comb_body(ge_ref, be_ref, ab_ref, *refs[:len(parts)])

    return pl.pallas_call(
        body,
        grid=(1,),
        in_specs=[pl.BlockSpec((2, HH), lambda e: (0, 0)) for _ in parts]
        + [pl.BlockSpec((1, HH), lambda e: (0, 0)),
           pl.BlockSpec((1, HH), lambda e: (0, 0))],
        out_specs=pl.BlockSpec((2, HH), lambda e: (0, 0)),
        out_shape=jax.ShapeDtypeStruct((2, HH), _F32),
    )(*parts, ge, be)


_BE2 = 2000


def _mlp_body(s_ref, g_ref, wn_ref, wd_ref, ab_ref,
              we2_ref, be2_ref, wmt_ref, bm_ref, wx1_ref, bx1_ref, wx2t_ref,
              m_ref, t16_ref):
    out, xd = _edge_out(s_ref[...], g_ref[...], wn_ref[...], wd_ref[...])
    out = jnp.maximum(out * ab_ref[0:1, :] + ab_ref[1:2, :], 0.0)
    out2 = jnp.maximum(
        jnp.dot(out, we2_ref[...], preferred_element_type=_F32,
                precision=_HI) + be2_ref[...], 0.0)
    logit = jnp.sum(out2 * wmt_ref[...], axis=1, keepdims=True) + bm_ref[0, 0]
    w = jax.nn.sigmoid(logit)
    m = out2 * w
    m_ref[...] = m
    t = jnp.maximum(
        jnp.dot(m, wx1_ref[...], preferred_element_type=_F32,
                precision=_HI) + bx1_ref[...], 0.0)
    xm = jnp.sum(t * wx2t_ref[...], axis=1, keepdims=True)
    trans = jnp.clip(xd * xm, -100.0, 100.0)
    nrow = trans.shape[0]
    t16_ref[...] = jnp.concatenate(
        [trans, jnp.ones((nrow, 1), _F32), jnp.zeros((nrow, XW - 5), _F32)],
        axis=1)


def _edge_mlp(s, g, wn, wd, ab, we2, be2, wmt, bm, wx1, bx1, wx2t):
    ne = s.shape[0]
    nblk = ne // _BE2
    return pl.pallas_call(
        _mlp_body,
        grid=(nblk,),
        in_specs=[
            pl.BlockSpec((_BE2, HH), lambda e: (e, 0)),
            pl.BlockSpec((_BE2, GW), lambda e: (e, 0)),
            pl.BlockSpec((1, HH), lambda e: (0, 0)),
            pl.BlockSpec((1, HH), lambda e: (0, 0)),
            pl.BlockSpec((2, HH), lambda e: (0, 0)),
            pl.BlockSpec((HH, HH), lambda e: (0, 0)),
            pl.BlockSpec((1, HH), lambda e: (0, 0)),
            pl.BlockSpec((1, HH), lambda e: (0, 0)),
            pl.BlockSpec((1, 1), lambda e: (0, 0)),
            pl.BlockSpec((HH, HH), lambda e: (0, 0)),
            pl.BlockSpec((1, HH), lambda e: (0, 0)),
            pl.BlockSpec((1, HH), lambda e: (0, 0)),
        ],
        out_specs=[
            pl.BlockSpec((_BE2, HH), lambda e: (e, 0)),
            pl.BlockSpec((_BE2, XW), lambda e: (e, 0)),
        ],
        out_shape=[
            jax.ShapeDtypeStruct((ne, HH), _F32),
            jax.ShapeDtypeStruct((ne, XW), _F32),
        ],
    )(s, g, wn, wd, ab, we2, be2, wmt, bm, wx1, bx1, wx2t)


_NSTRIPE = N // 16
_NROUND = _NSTRIPE // NS
_NXTRA = _NSTRIPE - _NROUND * NS


def _sc_scatter(src_width, ne):
    epw = ne // NW
    nfull = epw // CH
    tail = epw - nfull * CH
    inflate = src_width != HH
    mesh = plsc.VectorSubcoreMesh(core_axis_name="c", subcore_axis_name="s",
                                  num_cores=NC, num_subcores=NS)

    scratch = [
        pltpu.MemorySpace.VMEM_SHARED((N, HH), _F32),
        pltpu.VMEM((CH,), jnp.int32),
        pltpu.VMEM((CH, src_width), _F32),
        pltpu.VMEM((max(tail, 1),), jnp.int32),
        pltpu.VMEM((max(tail, 1), src_width), _F32),
    ]
    if inflate:
        scratch += [pltpu.VMEM((CH, HH), _F32),
                    pltpu.VMEM((max(tail, 1), HH), _F32)]

    @functools.partial(
        pl.kernel,
        out_type=jax.ShapeDtypeStruct((NC, N, HH), _F32),
        mesh=mesh,
        scratch_types=scratch,
    )
    def scatter_kernel(rows_hbm, i_hbm, acc_hbm, acc_sh, iv, rb, ivt, rbt,
                       *inf):
        cid = lax.axis_index("c")
        sid = lax.axis_index("s")
        wid = sid * NC + cid
        zv = jnp.zeros((16,), _F32)
        zbuf = inf[0] if inflate else rb

        @pl.loop(0, 16)
        def _zm(r):
            for c in range(HH // 16):
                zbuf[r, pl.ds(c * 16, 16)] = zv

        @pl.loop(0, _NROUND)
        def _clear(k):
            off = (k * NS + sid) * 16
            pltpu.sync_copy(zbuf.at[pl.ds(0, 16)], acc_sh.at[pl.ds(off, 16)])

        @pl.when(sid < _NXTRA)
        def _():
            off = (_NROUND * NS + sid) * 16
            pltpu.sync_copy(zbuf.at[pl.ds(0, 16)], acc_sh.at[pl.ds(off, 16)])

        if inflate:
            @pl.loop(16, CH)
            def _zrest(r):
                for c in range(HH // 16):
                    inf[0][r, pl.ds(c * 16, 16)] = zv

            @pl.loop(0, tail)
            def _zrt(r):
                for c in range(HH // 16):
                    inf[1][r, pl.ds(c * 16, 16)] = zv

        plsc.subcore_barrier()

        base = wid * epw

        def _do_chunk(off, n, ivb, rbb, ibuf):
            pltpu.sync_copy(i_hbm.at[pl.ds(off, n)], ivb)
            pltpu.sync_copy(rows_hbm.at[pl.ds(off, n)], rbb)
            if inflate:
                @pl.loop(0, n)
                def _inf(r):
                    ibuf[r, pl.ds(0, src_width)] = rbb[r, pl.ds(0, src_width)]

                pltpu.sync_copy(ibuf, acc_sh.at[ivb], add=True)
            else:
                pltpu.sync_copy(rbb, acc_sh.at[ivb], add=True)

        @pl.loop(0, nfull)
        def _chunk(k):
            _do_chunk(base + k * CH, CH, iv, rb, inf[0] if inflate else None)

        if tail:
            _do_chunk(base + nfull * CH, tail, ivt, rbt,
                      inf[1] if inflate else None)

        plsc.subcore_barrier()

        @pl.loop(0, _NROUND)
        def _flush(k):
            off = (k * NS + sid) * 16
            pltpu.sync_copy(acc_sh.at[pl.ds(off, 16)],
                            acc_hbm.at[cid, pl.ds(off, 16)])

        @pl.when(sid < _NXTRA)
        def _():
            off = (_NROUND * NS + sid) * 16
            pltpu.sync_copy(acc_sh.at[pl.ds(off, 16)],
                            acc_hbm.at[cid, pl.ds(off, 16)])

    return scatter_kernel


_BN_NODE = 2000


def _node1_body(h_ref, aggp_ref, na_ref, w1_ref, w2_ref, wc_ref, bh1_ref,
                gh_ref, bh_ref, hh_ref, ab2_ref, acc_ref):
    pid = pl.program_id(0)

    @pl.when(pid == 0)
    def _():
        acc_ref[...] = jnp.zeros_like(acc_ref)

    agg = aggp_ref[0]
    for k in range(1, NPART):
        agg = agg + aggp_ref[k]
    na = na_ref[...]
    hh = (jnp.dot(h_ref[...], w1_ref[...], preferred_element_type=_F32,
                  precision=_HI)
          + jnp.dot(agg, w2_ref[...], preferred_element_type=_F32,
                    precision=_HI)
          + na[:, 0:1] * wc_ref[0:1, :] + na[:, 1:2] * wc_ref[1:2, :]
          + bh1_ref[...])
    hh_ref[...] = hh
    acc_ref[0:1, :] += jnp.sum(hh, axis=0, keepdims=True)
    acc_ref[1:2, :] += jnp.sum(hh * hh, axis=0, keepdims=True)

    @pl.when(pid == pl.num_programs(0) - 1)
    def _():
        mu = acc_ref[0:1, :] * (1.0 / N)
        var = acc_ref[1:2, :] * (1.0 / N) - mu * mu
        a = gh_ref[...] * lax.rsqrt(var + 1e-5)
        b = bh_ref[...] - mu * a
        ab2_ref[...] = jnp.concatenate([a, b], axis=0)


def _node1(h, aggp, node_attr, w1, w2, wc, bh1, gh, bh):
    nblk = N // _BN_NODE
    return pl.pallas_call(
        _node1_body,
        grid=(nblk,),
        in_specs=[
            pl.BlockSpec((_BN_NODE, D), lambda n: (n, 0)),
            pl.BlockSpec((NPART, _BN_NODE, HH), lambda n: (0, n, 0)),
            pl.BlockSpec((_BN_NODE, 2), lambda n: (n, 0)),
            pl.BlockSpec((D, HH), lambda n: (0, 0)),
            pl.BlockSpec((HH, HH), lambda n: (0, 0)),
            pl.BlockSpec((2, HH), lambda n: (0, 0)),
            pl.BlockSpec((1, HH), lambda n: (0, 0)),
            pl.BlockSpec((1, HH), lambda n: (0, 0)),
            pl.BlockSpec((1, HH), lambda n: (0, 0)),
        ],
        out_specs=[
            pl.BlockSpec((_BN_NODE, HH), lambda n: (n, 0)),
            pl.BlockSpec((2, HH), lambda n: (0, 0)),
        ],
        out_shape=[
            jax.ShapeDtypeStruct((N, HH), _F32),
            jax.ShapeDtypeStruct((2, HH), _F32),
        ],
        scratch_shapes=[pltpu.VMEM((2, HH), _F32)],
    )(h, aggp, node_attr, w1, w2, wc, bh1, gh, bh)


def _node2_body(hh_ref, ab2_ref, h_ref, wh2_ref, bh2_ref, x_ref, tsp_ref,
                hn_ref, xn_ref):
    hh = jnp.maximum(hh_ref[...] * ab2_ref[0:1, :] + ab2_ref[1:2, :], 0.0)
    hn_ref[...] = (h_ref[...]
                   + jnp.dot(hh, wh2_ref[...], preferred_element_type=_F32,
                             precision=_HI) + bh2_ref[...])
    ts = tsp_ref[0]
    for k in range(1, NPART):
        ts = ts + tsp_ref[k]
    cnt = jnp.maximum(ts[:, 4:5], 1.0)
    xn_ref[...] = x_ref[...] + ts[:, 0:4] / cnt


def _node2(hh, ab2, h, wh2, bh2, x, tsp):
    nblk = N // _BN_NODE
    return pl.pallas_call(
        _node2_body,
        grid=(nblk,),
        in_specs=[
            pl.BlockSpec((_BN_NODE, HH), lambda n: (n, 0)),
            pl.BlockSpec((2, HH), lambda n: (0, 0)),
            pl.BlockSpec((_BN_NODE, D), lambda n: (n, 0)),
            pl.BlockSpec((HH, D), lambda n: (0, 0)),
            pl.BlockSpec((1, D), lambda n: (0, 0)),
            pl.BlockSpec((_BN_NODE, 4), lambda n: (n, 0)),
            pl.BlockSpec((NPART, _BN_NODE, HH), lambda n: (0, n, 0)),
        ],
        out_specs=[
            pl.BlockSpec((_BN_NODE, D), lambda n: (n, 0)),
            pl.BlockSpec((_BN_NODE, 4), lambda n: (n, 0)),
        ],
        out_shape=[
            jax.ShapeDtypeStruct((N, D), _F32),
            jax.ShapeDtypeStruct((N, 4), _F32),
        ],
    )(hh, ab2, h, wh2, bh2, x, tsp)


def kernel(h, x, edges, node_attr, we1, ge, be, we2, be2, wm, bm, wh1, bh1,
           gh, bh, wh2, bh2, wx1, bx1, wx2):
    ivec = edges[0]
    jvec = edges[1]
    w1a = we1[:D]
    w1b = we1[D:2 * D]
    wn = we1[2 * D].reshape(1, HH)
    wd = we1[2 * D + 1].reshape(1, HH)

    rptab, rqtab = _prep(h, x, w1a, w1b)

    ivs = [lax.slice_in_dim(ivec, k * ECH, (k + 1) * ECH)
           for k in range(NCHUNK)]
    jvs = [lax.slice_in_dim(jvec, k * ECH, (k + 1) * ECH)
           for k in range(NCHUNK)]

    ss = [_sc_gather(rptab, rqtab, ivs[k], jvs[k]) for k in range(NCHUNK)]
    parts = [_edge_stats(ss[k][0], ss[k][1], wn, wd) for k in range(NCHUNK)]
    ab = _stats_combine(parts, ge.reshape(1, HH), be.reshape(1, HH))

    ms, t16s = [], []
    for k in range(NCHUNK):
        mk, tk = _edge_mlp(ss[k][0], ss[k][1], wn, wd, ab, we2,
                           be2.reshape(1, HH), wm.reshape(1, HH),
                           bm.reshape(1, 1), wx1, bx1.reshape(1, HH),
                           wx2.reshape(1, HH))
        ms.append(mk)
        t16s.append(tk)

    aggps = [_sc_scatter(HH, ECH)(ms[k], ivs[k]) for k in range(NCHUNK)]
    tsps = [_sc_scatter(XW, ECH)(t16s[k], ivs[k]) for k in range(NCHUNK)]

    aggp = jnp.concatenate(aggps, axis=0)
    tsp = jnp.concatenate(tsps, axis=0)
    m = jnp.concatenate(ms, axis=0)

    hhpre, ab2 = _node1(h, aggp, node_attr, wh1[:D], wh1[D:2 * D],
                        wh1[2 * D:], bh1.reshape(1, HH), gh.reshape(1, HH),
                        bh.reshape(1, HH))
    h_new, x_new = _node2(hhpre, ab2, h, wh2, bh2.reshape(1, D), x, tsp)
    return (h_new, x_new, m)


# speedup vs baseline: 1.3966x; 1.1126x over previous
import functools

import jax
import jax.numpy as jnp
from jax import lax
from jax.experimental import pallas as pl
from jax.experimental.pallas import tpu as pltpu
from jax.experimental.pallas import tpu_sc as plsc

N = 10000
E = 320000
D = 128
HH = 128
RW = 256
NADD = 8
GW = 16
XW = 16

NC = 2
NS = 16
NW = NC * NS
CH = 128

NCHUNK = 2
ECH = E // NCHUNK
NPART = NC * NCHUNK

_HI = lax.Precision.DEFAULT
_F32 = jnp.float32


def _psi(v):
    return jnp.sign(v) * jnp.log1p(jnp.abs(v))


_BN_PREP = 2000


def _prep_body(h_ref, x_ref, w1a_ref, w1b_ref, rp_ref, rq_ref):
    hb = h_ref[...]
    xb = x_ref[...]
    p = jnp.dot(hb, w1a_ref[...], preferred_element_type=_F32,
                precision=_HI)
    q = jnp.dot(hb, w1b_ref[...], preferred_element_type=_F32,
                precision=_HI)
    n2 = (xb[:, 0:1] ** 2 - xb[:, 1:2] ** 2 - xb[:, 2:3] ** 2
          - xb[:, 3:4] ** 2)
    pad = jnp.zeros((xb.shape[0], RW - HH - 5), _F32)
    rp_ref[...] = jnp.concatenate([p, xb, n2, pad], axis=1)
    rq_ref[...] = jnp.concatenate([q, -xb, n2, pad], axis=1)


def _prep(h, x, w1a, w1b):
    nblk = N // _BN_PREP
    return pl.pallas_call(
        _prep_body,
        grid=(nblk,),
        in_specs=[
            pl.BlockSpec((_BN_PREP, D), lambda n: (n, 0)),
            pl.BlockSpec((_BN_PREP, 4), lambda n: (n, 0)),
            pl.BlockSpec((D, HH), lambda n: (0, 0)),
            pl.BlockSpec((D, HH), lambda n: (0, 0)),
        ],
        out_specs=[
            pl.BlockSpec((_BN_PREP, RW), lambda n: (n, 0)),
            pl.BlockSpec((_BN_PREP, RW), lambda n: (n, 0)),
        ],
        out_shape=[
            jax.ShapeDtypeStruct((N, RW), _F32),
            jax.ShapeDtypeStruct((N, RW), _F32),
        ],
    )(h, x, w1a, w1b)


GCH = 64


def _sc_gather(rptab, rqtab, ivec, jvec):
    ne = ivec.shape[0]
    epw = ne // NW
    nfull = epw // GCH
    npair = nfull // 2
    nodd = nfull - npair * 2
    tail = epw - nfull * GCH
    mesh = plsc.VectorSubcoreMesh(core_axis_name="c", subcore_axis_name="s",
                                  num_cores=NC, num_subcores=NS)

    @functools.partial(
        pl.kernel,
        out_type=[
            jax.ShapeDtypeStruct((ne, HH), _F32),
            jax.ShapeDtypeStruct((ne, GW), _F32),
        ],
        mesh=mesh,
        scratch_types=[
            pltpu.VMEM((GCH,), jnp.int32),
            pltpu.VMEM((GCH,), jnp.int32),
            pltpu.VMEM((GCH, RW), _F32),
            pltpu.VMEM((GCH, RW), _F32),
            pltpu.VMEM((GCH, GW), _F32),
            pltpu.VMEM((GCH,), jnp.int32),
            pltpu.VMEM((GCH,), jnp.int32),
            pltpu.VMEM((GCH, RW), _F32),
            pltpu.VMEM((GCH, RW), _F32),
            pltpu.VMEM((GCH, GW), _F32),
            pltpu.VMEM((max(tail, 1),), jnp.int32),
            pltpu.VMEM((max(tail, 1),), jnp.int32),
            pltpu.VMEM((max(tail, 1), RW), _F32),
            pltpu.VMEM((max(tail, 1), RW), _F32),
            pltpu.VMEM((max(tail, 1), GW), _F32),
            pltpu.SemaphoreType.DMA,
            pltpu.SemaphoreType.DMA,
            pltpu.SemaphoreType.DMA,
            pltpu.SemaphoreType.DMA,
        ],
    )
    def gather_kernel(rp_hbm, rq_hbm, i_hbm, j_hbm, s_hbm, g_hbm,
                      iv0, jv0, bp0, bq0, bg0, iv1, jv1, bp1, bq1, bg1,
                      ivt, jvt, bpt, bqt, bgt, semp0, semq0, semp1, semq1):
        wid = lax.axis_index("s") * NC + lax.axis_index("c")
        base = wid * epw
        gsl = pl.ds(HH, 16)

        def issue(off, n, iv, jv, bp, bq, semp, semq):
            pltpu.sync_copy(i_hbm.at[pl.ds(off, n)], iv)
            pltpu.sync_copy(j_hbm.at[pl.ds(off, n)], jv)
            cp = pltpu.async_copy(rp_hbm.at[iv], bp, semp)
            cq = pltpu.async_copy(rq_hbm.at[jv], bq, semq)
            return cp, cq

        def process(off, n, bp, bq, bg, cp, cq):
            cp.wait()
            cq.wait()

            @pl.loop(0, n)
            def _row(r):
                for c in range(NADD):
                    sl = pl.ds(c * 16, 16)
                    bp[r, sl] = bp[r, sl] + bq[r, sl]
                bg[r, pl.ds(0, 16)] = bp[r, gsl] + bq[r, gsl]

            pltpu.sync_copy(bp.at[:, pl.ds(0, HH)], s_hbm.at[pl.ds(off, n)])
            pltpu.sync_copy(bg, g_hbm.at[pl.ds(off, n)])

        @pl.loop(0, npair)
        def _pair(p):
            off0 = base + (2 * p) * GCH
            off1 = off0 + GCH
            h0 = issue(off0, GCH, iv0, jv0, bp0, bq0, semp0, semq0)
            h1 = issue(off1, GCH, iv1, jv1, bp1, bq1, semp1, semq1)
            process(off0, GCH, bp0, bq0, bg0, *h0)
            process(off1, GCH, bp1, bq1, bg1, *h1)

        if nodd:
            offo = base + (npair * 2) * GCH
            ho = issue(offo, GCH, iv0, jv0, bp0, bq0, semp0, semq0)
            process(offo, GCH, bp0, bq0, bg0, *ho)

        if tail:
            offt = base + nfull * GCH
            ht = issue(offt, tail, ivt, jvt, bpt, bqt, semp1, semq1)
            process(offt, tail, bpt, bqt, bgt, *ht)

    return gather_kernel(rptab, rqtab, ivec, jvec)


def _edge_out(s0, g, wn, wd):
    xd = g[:, 0:4]
    n2s = g[:, 4:5]
    norms = (xd[:, 0:1] ** 2 - xd[:, 1:2] ** 2 - xd[:, 2:3] ** 2
             - xd[:, 3:4] ** 2)
    dots = 0.5 * (n2s - norms)
    out = s0 + _psi(norms) * wn + _psi(dots) * wd
    return out, xd


_BE1 = 3200


def _stats_body(s_ref, g_ref, wn_ref, wd_ref, p_ref, acc_ref):
    pid = pl.program_id(0)

    @pl.when(pid == 0)
    def _():
        acc_ref[...] = jnp.zeros_like(acc_ref)

    out, _ = _edge_out(s_ref[...], g_ref[...], wn_ref[...], wd_ref[...])
    acc_ref[0:1, :] += jnp.sum(out, axis=0, keepdims=True)
    acc_ref[1:2, :] += jnp.sum(out * out, axis=0, keepdims=True)

    @pl.when(pid == pl.num_programs(0) - 1)
    def _():
        p_ref[...] = acc_ref[...]


def _edge_stats(s, g, wn, wd):
    ne = s.shape[0]
    nblk = ne // _BE1
    return pl.pallas_call(
        _stats_body,
        grid=(nblk,),
        in_specs=[
            pl.BlockSpec((_BE1, HH), lambda e: (e, 0)),
            pl.BlockSpec((_BE1, GW), lambda e: (e, 0)),
            pl.BlockSpec((1, HH), lambda e: (0, 0)),
            pl.BlockSpec((1, HH), lambda e: (0, 0)),
        ],
        out_specs=pl.BlockSpec((2, HH), lambda e: (0, 0)),
        out_shape=jax.ShapeDtypeStruct((2, HH), _F32),
        scratch_shapes=[pltpu.VMEM((2, HH), _F32)],
    )(s, g, wn, wd)


def _comb_body(ge_ref, be_ref, ab_ref, *p_refs):
    s = p_refs[0][...]
    for p in p_refs[1:]:
        s = s + p[...]
    mu = s[0:1, :] * (1.0 / E)
    var = s[1:2, :] * (1.0 / E) - mu * mu
    a = ge_ref[...] * lax.rsqrt(var + 1e-5)
    b = be_ref[...] - mu * a
    ab_ref[...] = jnp.concatenate([a, b], axis=0)


def _stats_combine(parts, ge, be):
    def body(*refs):
        ge_ref, be_ref = refs[len(parts)], refs[len(parts) + 1]
        ab_ref = refs[len(parts) + 2]
        _comb_body(ge_ref, be_ref, ab_ref, *refs[:len(parts)])

    return pl.pallas_call(
        body,
        grid=(1,),
        in_specs=[pl.BlockSpec((2, HH), lambda e: (0, 0)) for _ in parts]
        + [pl.BlockSpec((1, HH), lambda e: (0, 0)),
           pl.BlockSpec((1, HH), lambda e: (0, 0))],
        out_specs=pl.BlockSpec((2, HH), lambda e: (0, 0)),
        out_shape=jax.ShapeDtypeStruct((2, HH), _F32),
    )(*parts, ge, be)


_BE2 = 2000


def _mlp_body(s_ref, g_ref, wn_ref, wd_ref, ab_ref,
              we2_ref, be2_ref, wmt_ref, bm_ref, wx1_ref, bx1_ref, wx2t_ref,
              m_ref, t16_ref):
    out, xd = _edge_out(s_ref[...], g_ref[...], wn_ref[...], wd_ref[...])
    out = jnp.maximum(out * ab_ref[0:1, :] + ab_ref[1:2, :], 0.0)
    out2 = jnp.maximum(
        jnp.dot(out, we2_ref[...], preferred_element_type=_F32,
                precision=_HI) + be2_ref[...], 0.0)
    logit = jnp.sum(out2 * wmt_ref[...], axis=1, keepdims=True) + bm_ref[0, 0]
    w = jax.nn.sigmoid(logit)
    m = out2 * w
    m_ref[...] = m
    t = jnp.maximum(
        jnp.dot(m, wx1_ref[...], preferred_element_type=_F32,
                precision=_HI) + bx1_ref[...], 0.0)
    xm = jnp.sum(t * wx2t_ref[...], axis=1, keepdims=True)
    trans = jnp.clip(xd * xm, -100.0, 100.0)
    nrow = trans.shape[0]
    t16_ref[...] = jnp.concatenate(
        [trans, jnp.ones((nrow, 1), _F32), jnp.zeros((nrow, XW - 5), _F32)],
        axis=1)


def _edge_mlp(s, g, wn, wd, ab, we2, be2, wmt, bm, wx1, bx1, wx2t):
    ne = s.shape[0]
    nblk = ne // _BE2
    return pl.pallas_call(
        _mlp_body,
        grid=(nblk,),
        in_specs=[
            pl.BlockSpec((_BE2, HH), lambda e: (e, 0)),
            pl.BlockSpec((_BE2, GW), lambda e: (e, 0)),
            pl.BlockSpec((1, HH), lambda e: (0, 0)),
            pl.BlockSpec((1, HH), lambda e: (0, 0)),
            pl.BlockSpec((2, HH), lambda e: (0, 0)),
            pl.BlockSpec((HH, HH), lambda e: (0, 0)),
            pl.BlockSpec((1, HH), lambda e: (0, 0)),
            pl.BlockSpec((1, HH), lambda e: (0, 0)),
            pl.BlockSpec((1, 1), lambda e: (0, 0)),
            pl.BlockSpec((HH, HH), lambda e: (0, 0)),
            pl.BlockSpec((1, HH), lambda e: (0, 0)),
            pl.BlockSpec((1, HH), lambda e: (0, 0)),
        ],
        out_specs=[
            pl.BlockSpec((_BE2, HH), lambda e: (e, 0)),
            pl.BlockSpec((_BE2, XW), lambda e: (e, 0)),
        ],
        out_shape=[
            jax.ShapeDtypeStruct((ne, HH), _F32),
            jax.ShapeDtypeStruct((ne, XW), _F32),
        ],
    )(s, g, wn, wd, ab, we2, be2, wmt, bm, wx1, bx1, wx2t)


_NSTRIPE = N // 16
_NROUND = _NSTRIPE // NS
_NXTRA = _NSTRIPE - _NROUND * NS


def _sc_scatter(src_width, ne, ch=CH):
    epw = ne // NW
    nfull = epw // ch
    npair = nfull // 2
    nodd = nfull - npair * 2
    tail = epw - nfull * ch
    inflate = src_width != HH
    mesh = plsc.VectorSubcoreMesh(core_axis_name="c", subcore_axis_name="s",
                                  num_cores=NC, num_subcores=NS)

    scratch = [
        pltpu.MemorySpace.VMEM_SHARED((N, HH), _F32),
        pltpu.VMEM((ch,), jnp.int32),
        pltpu.VMEM((ch, src_width), _F32),
        pltpu.VMEM((ch,), jnp.int32),
        pltpu.VMEM((ch, src_width), _F32),
        pltpu.VMEM((max(tail, 1),), jnp.int32),
        pltpu.VMEM((max(tail, 1), src_width), _F32),
        pltpu.SemaphoreType.DMA,
        pltpu.SemaphoreType.DMA,
        pltpu.SemaphoreType.DMA,
        pltpu.SemaphoreType.DMA,
    ]
    if inflate:
        scratch += [pltpu.VMEM((ch, HH), _F32),
                    pltpu.VMEM((max(tail, 1), HH), _F32)]

    @functools.partial(
        pl.kernel,
        out_type=jax.ShapeDtypeStruct((NC, N, HH), _F32),
        mesh=mesh,
        scratch_types=scratch,
    )
    def scatter_kernel(rows_hbm, i_hbm, acc_hbm, acc_sh, iv0, rb0, iv1, rb1,
                       ivt, rbt, semi0, semr0, semi1, semr1, *inf):
        cid = lax.axis_index("c")
        sid = lax.axis_index("s")
        wid = sid * NC + cid
        zv = jnp.zeros((16,), _F32)
        zbuf = inf[0] if inflate else rb0

        @pl.loop(0, 16)
        def _zm(r):
            for c in range(HH // 16):
                zbuf[r, pl.ds(c * 16, 16)] = zv

        @pl.loop(0, _NROUND)
        def _clear(k):
            off = (k * NS + sid) * 16
            pltpu.sync_copy(zbuf.at[pl.ds(0, 16)], acc_sh.at[pl.ds(off, 16)])

        @pl.when(sid < _NXTRA)
        def _():
            off = (_NROUND * NS + sid) * 16
            pltpu.sync_copy(zbuf.at[pl.ds(0, 16)], acc_sh.at[pl.ds(off, 16)])

        if inflate:
            @pl.loop(16, ch)
            def _zrest(r):
                for c in range(HH // 16):
                    inf[0][r, pl.ds(c * 16, 16)] = zv

            @pl.loop(0, tail)
            def _zrt(r):
                for c in range(HH // 16):
                    inf[1][r, pl.ds(c * 16, 16)] = zv

        plsc.subcore_barrier()

        base = wid * epw

        def issue(off, n, ivb, rbb, semi, semr):
            ci = pltpu.async_copy(i_hbm.at[pl.ds(off, n)], ivb, semi)
            cr = pltpu.async_copy(rows_hbm.at[pl.ds(off, n)], rbb, semr)
            return ci, cr

        def scat(n, ivb, rbb, ibuf, ci, cr):
            ci.wait()
            cr.wait()
            if inflate:
                @pl.loop(0, n)
                def _inf(r):
                    ibuf[r, pl.ds(0, src_width)] = rbb[r, pl.ds(0, src_width)]

                pltpu.sync_copy(ibuf, acc_sh.at[ivb], add=True)
            else:
                pltpu.sync_copy(rbb, acc_sh.at[ivb], add=True)

        ib0 = inf[0] if inflate else None
        ib1 = inf[0] if inflate else None

        @pl.loop(0, npair)
        def _pair(p):
            off0 = base + (2 * p) * ch
            h0 = issue(off0, ch, iv0, rb0, semi0, semr0)
            h1 = issue(off0 + ch, ch, iv1, rb1, semi1, semr1)
            scat(ch, iv0, rb0, ib0, *h0)
            scat(ch, iv1, rb1, ib1, *h1)

        if nodd:
            offo = base + (npair * 2) * ch
            ho = issue(offo, ch, iv0, rb0, semi0, semr0)
            scat(ch, iv0, rb0, ib0, *ho)

        if tail:
            offt = base + nfull * ch
            ht = issue(offt, tail, ivt, rbt, semi1, semr1)
            scat(tail, ivt, rbt, inf[1] if inflate else None, *ht)

        plsc.subcore_barrier()

        @pl.loop(0, _NROUND)
        def _flush(k):
            off = (k * NS + sid) * 16
            pltpu.sync_copy(acc_sh.at[pl.ds(off, 16)],
                            acc_hbm.at[cid, pl.ds(off, 16)])

        @pl.when(sid < _NXTRA)
        def _():
            off = (_NROUND * NS + sid) * 16
            pltpu.sync_copy(acc_sh.at[pl.ds(off, 16)],
                            acc_hbm.at[cid, pl.ds(off, 16)])

    return scatter_kernel


_BN_NODE = 2000


def _node1_body(h_ref, aggp_ref, na_ref, w1_ref, w2_ref, wc_ref, bh1_ref,
                gh_ref, bh_ref, hh_ref, ab2_ref, acc_ref):
    pid = pl.program_id(0)

    @pl.when(pid == 0)
    def _():
        acc_ref[...] = jnp.zeros_like(acc_ref)

    agg = aggp_ref[0]
    for k in range(1, NPART):
        agg = agg + aggp_ref[k]
    na = na_ref[...]
    hh = (jnp.dot(h_ref[...], w1_ref[...], preferred_element_type=_F32,
                  precision=_HI)
          + jnp.dot(agg, w2_ref[...], preferred_element_type=_F32,
                    precision=_HI)
          + na[:, 0:1] * wc_ref[0:1, :] + na[:, 1:2] * wc_ref[1:2, :]
          + bh1_ref[...])
    hh_ref[...] = hh
    acc_ref[0:1, :] += jnp.sum(hh, axis=0, keepdims=True)
    acc_ref[1:2, :] += jnp.sum(hh * hh, axis=0, keepdims=True)

    @pl.when(pid == pl.num_programs(0) - 1)
    def _():
        mu = acc_ref[0:1, :] * (1.0 / N)
        var = acc_ref[1:2, :] * (1.0 / N) - mu * mu
        a = gh_ref[...] * lax.rsqrt(var + 1e-5)
        b = bh_ref[...] - mu * a
        ab2_ref[...] = jnp.concatenate([a, b], axis=0)


def _node1(h, aggp, node_attr, w1, w2, wc, bh1, gh, bh):
    nblk = N // _BN_NODE
    return pl.pallas_call(
        _node1_body,
        grid=(nblk,),
        in_specs=[
            pl.BlockSpec((_BN_NODE, D), lambda n: (n, 0)),
            pl.BlockSpec((NPART, _BN_NODE, HH), lambda n: (0, n, 0)),
            pl.BlockSpec((_BN_NODE, 2), lambda n: (n, 0)),
            pl.BlockSpec((D, HH), lambda n: (0, 0)),
            pl.BlockSpec((HH, HH), lambda n: (0, 0)),
            pl.BlockSpec((2, HH), lambda n: (0, 0)),
            pl.BlockSpec((1, HH), lambda n: (0, 0)),
            pl.BlockSpec((1, HH), lambda n: (0, 0)),
            pl.BlockSpec((1, HH), lambda n: (0, 0)),
        ],
        out_specs=[
            pl.BlockSpec((_BN_NODE, HH), lambda n: (n, 0)),
            pl.BlockSpec((2, HH), lambda n: (0, 0)),
        ],
        out_shape=[
            jax.ShapeDtypeStruct((N, HH), _F32),
            jax.ShapeDtypeStruct((2, HH), _F32),
        ],
        scratch_shapes=[pltpu.VMEM((2, HH), _F32)],
    )(h, aggp, node_attr, w1, w2, wc, bh1, gh, bh)


def _node2_body(hh_ref, ab2_ref, h_ref, wh2_ref, bh2_ref, x_ref, tsp_ref,
                hn_ref, xn_ref):
    hh = jnp.maximum(hh_ref[...] * ab2_ref[0:1, :] + ab2_ref[1:2, :], 0.0)
    hn_ref[...] = (h_ref[...]
                   + jnp.dot(hh, wh2_ref[...], preferred_element_type=_F32,
                             precision=_HI) + bh2_ref[...])
    ts = tsp_ref[0]
    for k in range(1, NPART):
        ts = ts + tsp_ref[k]
    cnt = jnp.maximum(ts[:, 4:5], 1.0)
    xn_ref[...] = x_ref[...] + ts[:, 0:4] / cnt


def _node2(hh, ab2, h, wh2, bh2, x, tsp):
    nblk = N // _BN_NODE
    return pl.pallas_call(
        _node2_body,
        grid=(nblk,),
        in_specs=[
            pl.BlockSpec((_BN_NODE, HH), lambda n: (n, 0)),
            pl.BlockSpec((2, HH), lambda n: (0, 0)),
            pl.BlockSpec((_BN_NODE, D), lambda n: (n, 0)),
            pl.BlockSpec((HH, D), lambda n: (0, 0)),
            pl.BlockSpec((1, D), lambda n: (0, 0)),
            pl.BlockSpec((_BN_NODE, 4), lambda n: (n, 0)),
            pl.BlockSpec((NPART, _BN_NODE, HH), lambda n: (0, n, 0)),
        ],
        out_specs=[
            pl.BlockSpec((_BN_NODE, D), lambda n: (n, 0)),
            pl.BlockSpec((_BN_NODE, 4), lambda n: (n, 0)),
        ],
        out_shape=[
            jax.ShapeDtypeStruct((N, D), _F32),
            jax.ShapeDtypeStruct((N, 4), _F32),
        ],
    )(hh, ab2, h, wh2, bh2, x, tsp)


def kernel(h, x, edges, node_attr, we1, ge, be, we2, be2, wm, bm, wh1, bh1,
           gh, bh, wh2, bh2, wx1, bx1, wx2):
    ivec = edges[0]
    jvec = edges[1]
    w1a = we1[:D]
    w1b = we1[D:2 * D]
    wn = we1[2 * D].reshape(1, HH)
    wd = we1[2 * D + 1].reshape(1, HH)

    rptab, rqtab = _prep(h, x, w1a, w1b)

    ivs = [lax.slice_in_dim(ivec, k * ECH, (k + 1) * ECH)
           for k in range(NCHUNK)]
    jvs = [lax.slice_in_dim(jvec, k * ECH, (k + 1) * ECH)
           for k in range(NCHUNK)]

    ss = [_sc_gather(rptab, rqtab, ivs[k], jvs[k]) for k in range(NCHUNK)]
    parts = [_edge_stats(ss[k][0], ss[k][1], wn, wd) for k in range(NCHUNK)]
    ab = _stats_combine(parts, ge.reshape(1, HH), be.reshape(1, HH))

    ms, t16s = [], []
    for k in range(NCHUNK):
        mk, tk = _edge_mlp(ss[k][0], ss[k][1], wn, wd, ab, we2,
                           be2.reshape(1, HH), wm.reshape(1, HH),
                           bm.reshape(1, 1), wx1, bx1.reshape(1, HH),
                           wx2.reshape(1, HH))
        ms.append(mk)
        t16s.append(tk)

    aggps = [_sc_scatter(HH, ECH)(ms[k], ivs[k]) for k in range(NCHUNK)]
    tsps = [_sc_scatter(XW, ECH, ch=64)(t16s[k], ivs[k])
            for k in range(NCHUNK)]

    aggp = jnp.concatenate(aggps, axis=0)
    tsp = jnp.concatenate(tsps, axis=0)
    m = jnp.concatenate(ms, axis=0)

    hhpre, ab2 = _node1(h, aggp, node_attr, wh1[:D], wh1[D:2 * D],
                        wh1[2 * D:], bh1.reshape(1, HH), gh.reshape(1, HH),
                        bh.reshape(1, HH))
    h_new, x_new = _node2(hhpre, ab2, h, wh2, bh2.reshape(1, D), x, tsp)
    return (h_new, x_new, m)


# speedup vs baseline: 1.4776x; 1.0580x over previous
import functools

import jax
import jax.numpy as jnp
from jax import lax
from jax.experimental import pallas as pl
from jax.experimental.pallas import tpu as pltpu
from jax.experimental.pallas import tpu_sc as plsc

N = 10000
E = 320000
D = 128
HH = 128
RW = 256
NADD = 8
GW = 16
XW = 16

NC = 2
NS = 16
NW = NC * NS
CH = 128

NCHUNK = 2
ECH = E // NCHUNK
NPART = NC * NCHUNK

_HI = lax.Precision.DEFAULT
_F32 = jnp.float32


def _psi(v):
    return jnp.sign(v) * jnp.log1p(jnp.abs(v))


_BN_PREP = 2000


def _prep_body(h_ref, x_ref, w1a_ref, w1b_ref, rp_ref, rq_ref):
    hb = h_ref[...]
    xb = x_ref[...]
    p = jnp.dot(hb, w1a_ref[...], preferred_element_type=_F32,
                precision=_HI)
    q = jnp.dot(hb, w1b_ref[...], preferred_element_type=_F32,
                precision=_HI)
    n2 = (xb[:, 0:1] ** 2 - xb[:, 1:2] ** 2 - xb[:, 2:3] ** 2
          - xb[:, 3:4] ** 2)
    pad = jnp.zeros((xb.shape[0], RW - HH - 5), _F32)
    rp_ref[...] = jnp.concatenate([p, xb, n2, pad], axis=1)
    rq_ref[...] = jnp.concatenate([q, -xb, n2, pad], axis=1)


def _prep(h, x, w1a, w1b):
    nblk = N // _BN_PREP
    return pl.pallas_call(
        _prep_body,
        grid=(nblk,),
        in_specs=[
            pl.BlockSpec((_BN_PREP, D), lambda n: (n, 0)),
            pl.BlockSpec((_BN_PREP, 4), lambda n: (n, 0)),
            pl.BlockSpec((D, HH), lambda n: (0, 0)),
            pl.BlockSpec((D, HH), lambda n: (0, 0)),
        ],
        out_specs=[
            pl.BlockSpec((_BN_PREP, RW), lambda n: (n, 0)),
            pl.BlockSpec((_BN_PREP, RW), lambda n: (n, 0)),
        ],
        out_shape=[
            jax.ShapeDtypeStruct((N, RW), _F32),
            jax.ShapeDtypeStruct((N, RW), _F32),
        ],
    )(h, x, w1a, w1b)


GCH = 128


def _sc_gather(rptab, rqtab, ivec, jvec):
    ne = ivec.shape[0]
    epw = ne // NW
    nfull = epw // GCH
    npair = nfull // 2
    nodd = nfull - npair * 2
    tail = epw - nfull * GCH
    mesh = plsc.VectorSubcoreMesh(core_axis_name="c", subcore_axis_name="s",
                                  num_cores=NC, num_subcores=NS)

    @functools.partial(
        pl.kernel,
        out_type=[
            jax.ShapeDtypeStruct((ne, HH), _F32),
            jax.ShapeDtypeStruct((ne, GW), _F32),
        ],
        mesh=mesh,
        scratch_types=[
            pltpu.VMEM((GCH,), jnp.int32),
            pltpu.VMEM((GCH,), jnp.int32),
            pltpu.VMEM((GCH, RW), _F32),
            pltpu.VMEM((GCH, GW), _F32),
            pltpu.VMEM((GCH,), jnp.int32),
            pltpu.VMEM((GCH,), jnp.int32),
            pltpu.VMEM((GCH, RW), _F32),
            pltpu.VMEM((GCH, GW), _F32),
            pltpu.VMEM((max(tail, 1),), jnp.int32),
            pltpu.VMEM((max(tail, 1),), jnp.int32),
            pltpu.VMEM((max(tail, 1), RW), _F32),
            pltpu.VMEM((max(tail, 1), GW), _F32),
            pltpu.SemaphoreType.DMA,
            pltpu.SemaphoreType.DMA,
            pltpu.SemaphoreType.DMA,
            pltpu.SemaphoreType.DMA,
        ],
    )
    def gather_kernel(rp_hbm, rq_hbm, i_hbm, j_hbm, s_hbm, g_hbm,
                      iv0, jv0, bp0, bg0, iv1, jv1, bp1, bg1,
                      ivt, jvt, bpt, bgt, semp0, semq0, semp1, semq1):
        wid = lax.axis_index("s") * NC + lax.axis_index("c")
        base = wid * epw
        gsl = pl.ds(HH, 16)

        def issue(off, n, iv, jv, bp, semp):
            pltpu.sync_copy(i_hbm.at[pl.ds(off, n)], iv)
            pltpu.sync_copy(j_hbm.at[pl.ds(off, n)], jv)
            return pltpu.async_copy(rp_hbm.at[iv], bp, semp)

        def add_issue(jv, bp, cp, semq):
            cp.wait()
            return pltpu.async_copy(rq_hbm.at[jv], bp, semq, add=True)

        def finish(off, n, bp, bg, cq):
            cq.wait()

            @pl.loop(0, n)
            def _row(r):
                bg[r, pl.ds(0, 16)] = bp[r, gsl]

            pltpu.sync_copy(bp.at[:, pl.ds(0, HH)], s_hbm.at[pl.ds(off, n)])
            pltpu.sync_copy(bg, g_hbm.at[pl.ds(off, n)])

        @pl.loop(0, npair)
        def _pair(p):
            off0 = base + (2 * p) * GCH
            off1 = off0 + GCH
            cp0 = issue(off0, GCH, iv0, jv0, bp0, semp0)
            cp1 = issue(off1, GCH, iv1, jv1, bp1, semp1)
            cq0 = add_issue(jv0, bp0, cp0, semq0)
            cq1 = add_issue(jv1, bp1, cp1, semq1)
            finish(off0, GCH, bp0, bg0, cq0)
            finish(off1, GCH, bp1, bg1, cq1)

        if nodd:
            offo = base + (npair * 2) * GCH
            cpo = issue(offo, GCH, iv0, jv0, bp0, semp0)
            cqo = add_issue(jv0, bp0, cpo, semq0)
            finish(offo, GCH, bp0, bg0, cqo)

        if tail:
            offt = base + nfull * GCH
            cpt = issue(offt, tail, ivt, jvt, bpt, semp1)
            cqt = add_issue(jvt, bpt, cpt, semq1)
            finish(offt, tail, bpt, bgt, cqt)

    return gather_kernel(rptab, rqtab, ivec, jvec)


def _edge_out(s0, g, wn, wd):
    xd = g[:, 0:4]
    n2s = g[:, 4:5]
    norms = (xd[:, 0:1] ** 2 - xd[:, 1:2] ** 2 - xd[:, 2:3] ** 2
             - xd[:, 3:4] ** 2)
    dots = 0.5 * (n2s - norms)
    out = s0 + _psi(norms) * wn + _psi(dots) * wd
    return out, xd


_BE1 = 3200


def _stats_body(s_ref, g_ref, wn_ref, wd_ref, p_ref, acc_ref):
    pid = pl.program_id(0)

    @pl.when(pid == 0)
    def _():
        acc_ref[...] = jnp.zeros_like(acc_ref)

    out, _ = _edge_out(s_ref[...], g_ref[...], wn_ref[...], wd_ref[...])
    acc_ref[0:1, :] += jnp.sum(out, axis=0, keepdims=True)
    acc_ref[1:2, :] += jnp.sum(out * out, axis=0, keepdims=True)

    @pl.when(pid == pl.num_programs(0) - 1)
    def _():
        p_ref[...] = acc_ref[...]


def _edge_stats(s, g, wn, wd):
    ne = s.shape[0]
    nblk = ne // _BE1
    return pl.pallas_call(
        _stats_body,
        grid=(nblk,),
        in_specs=[
            pl.BlockSpec((_BE1, HH), lambda e: (e, 0)),
            pl.BlockSpec((_BE1, GW), lambda e: (e, 0)),
            pl.BlockSpec((1, HH), lambda e: (0, 0)),
            pl.BlockSpec((1, HH), lambda e: (0, 0)),
        ],
        out_specs=pl.BlockSpec((2, HH), lambda e: (0, 0)),
        out_shape=jax.ShapeDtypeStruct((2, HH), _F32),
        scratch_shapes=[pltpu.VMEM((2, HH), _F32)],
    )(s, g, wn, wd)


def _comb_body(ge_ref, be_ref, ab_ref, *p_refs):
    s = p_refs[0][...]
    for p in p_refs[1:]:
        s = s + p[...]
    mu = s[0:1, :] * (1.0 / E)
    var = s[1:2, :] * (1.0 / E) - mu * mu
    a = ge_ref[...] * lax.rsqrt(var + 1e-5)
    b = be_ref[...] - mu * a
    ab_ref[...] = jnp.concatenate([a, b], axis=0)


def _stats_combine(parts, ge, be):
    def body(*refs):
        ge_ref, be_ref = refs[len(parts)], refs[len(parts) + 1]
        ab_ref = refs[len(parts) + 2]
        _comb_body(ge_ref, be_ref, ab_ref, *refs[:len(parts)])

    return pl.pallas_call(
        body,
        grid=(1,),
        in_specs=[pl.BlockSpec((2, HH), lambda e: (0, 0)) for _ in parts]
        + [pl.BlockSpec((1, HH), lambda e: (0, 0)),
           pl.BlockSpec((1, HH), lambda e: (0, 0))],
        out_specs=pl.BlockSpec((2, HH), lambda e: (0, 0)),
        out_shape=jax.ShapeDtypeStruct((2, HH), _F32),
    )(*parts, ge, be)


_BE2 = 2000


def _mlp_body(s_ref, g_ref, wn_ref, wd_ref, ab_ref,
              we2_ref, be2_ref, wmt_ref, bm_ref, wx1_ref, bx1_ref, wx2t_ref,
              m_ref, t16_ref):
    out, xd = _edge_out(s_ref[...], g_ref[...], wn_ref[...], wd_ref[...])
    out = jnp.maximum(out * ab_ref[0:1, :] + ab_ref[1:2, :], 0.0)
    out2 = jnp.maximum(
        jnp.dot(out, we2_ref[...], preferred_element_type=_F32,
                precision=_HI) + be2_ref[...], 0.0)
    logit = jnp.sum(out2 * wmt_ref[...], axis=1, keepdims=True) + bm_ref[0, 0]
    w = jax.nn.sigmoid(logit)
    m = out2 * w
    m_ref[...] = m
    t = jnp.maximum(
        jnp.dot(m, wx1_ref[...], preferred_element_type=_F32,
                precision=_HI) + bx1_ref[...], 0.0)
    xm = jnp.sum(t * wx2t_ref[...], axis=1, keepdims=True)
    trans = jnp.clip(xd * xm, -100.0, 100.0)
    nrow = trans.shape[0]
    t16_ref[...] = jnp.concatenate(
        [trans, jnp.ones((nrow, 1), _F32), jnp.zeros((nrow, XW - 5), _F32)],
        axis=1)


def _edge_mlp(s, g, wn, wd, ab, we2, be2, wmt, bm, wx1, bx1, wx2t):
    ne = s.shape[0]
    nblk = ne // _BE2
    return pl.pallas_call(
        _mlp_body,
        grid=(nblk,),
        in_specs=[
            pl.BlockSpec((_BE2, HH), lambda e: (e, 0)),
            pl.BlockSpec((_BE2, GW), lambda e: (e, 0)),
            pl.BlockSpec((1, HH), lambda e: (0, 0)),
            pl.BlockSpec((1, HH), lambda e: (0, 0)),
            pl.BlockSpec((2, HH), lambda e: (0, 0)),
            pl.BlockSpec((HH, HH), lambda e: (0, 0)),
            pl.BlockSpec((1, HH), lambda e: (0, 0)),
            pl.BlockSpec((1, HH), lambda e: (0, 0)),
            pl.BlockSpec((1, 1), lambda e: (0, 0)),
            pl.BlockSpec((HH, HH), lambda e: (0, 0)),
            pl.BlockSpec((1, HH), lambda e: (0, 0)),
            pl.BlockSpec((1, HH), lambda e: (0, 0)),
        ],
        out_specs=[
            pl.BlockSpec((_BE2, HH), lambda e: (e, 0)),
            pl.BlockSpec((_BE2, XW), lambda e: (e, 0)),
        ],
        out_shape=[
            jax.ShapeDtypeStruct((ne, HH), _F32),
            jax.ShapeDtypeStruct((ne, XW), _F32),
        ],
    )(s, g, wn, wd, ab, we2, be2, wmt, bm, wx1, bx1, wx2t)


_NSTRIPE = N // 16
_NROUND = _NSTRIPE // NS
_NXTRA = _NSTRIPE - _NROUND * NS


def _sc_scatter(src_width, ne, ch=CH):
    epw = ne // NW
    nfull = epw // ch
    npair = nfull // 2
    nodd = nfull - npair * 2
    tail = epw - nfull * ch
    inflate = src_width != HH
    mesh = plsc.VectorSubcoreMesh(core_axis_name="c", subcore_axis_name="s",
                                  num_cores=NC, num_subcores=NS)

    scratch = [
        pltpu.MemorySpace.VMEM_SHARED((N, HH), _F32),
        pltpu.VMEM((ch,), jnp.int32),
        pltpu.VMEM((ch, src_width), _F32),
        pltpu.VMEM((ch,), jnp.int32),
        pltpu.VMEM((ch, src_width), _F32),
        pltpu.VMEM((max(tail, 1),), jnp.int32),
        pltpu.VMEM((max(tail, 1), src_width), _F32),
        pltpu.SemaphoreType.DMA,
        pltpu.SemaphoreType.DMA,
        pltpu.SemaphoreType.DMA,
        pltpu.SemaphoreType.DMA,
    ]
    if inflate:
        scratch += [pltpu.VMEM((ch, HH), _F32),
                    pltpu.VMEM((max(tail, 1), HH), _F32)]

    @functools.partial(
        pl.kernel,
        out_type=jax.ShapeDtypeStruct((NC, N, HH), _F32),
        mesh=mesh,
        scratch_types=scratch,
    )
    def scatter_kernel(rows_hbm, i_hbm, acc_hbm, acc_sh, iv0, rb0, iv1, rb1,
                       ivt, rbt, semi0, semr0, semi1, semr1, *inf):
        cid = lax.axis_index("c")
        sid = lax.axis_index("s")
        wid = sid * NC + cid
        zv = jnp.zeros((16,), _F32)
        zbuf = inf[0] if inflate else rb0

        @pl.loop(0, 16)
        def _zm(r):
            for c in range(HH // 16):
                zbuf[r, pl.ds(c * 16, 16)] = zv

        @pl.loop(0, _NROUND)
        def _clear(k):
            off = (k * NS + sid) * 16
            pltpu.sync_copy(zbuf.at[pl.ds(0, 16)], acc_sh.at[pl.ds(off, 16)])

        @pl.when(sid < _NXTRA)
        def _():
            off = (_NROUND * NS + sid) * 16
            pltpu.sync_copy(zbuf.at[pl.ds(0, 16)], acc_sh.at[pl.ds(off, 16)])

        if inflate:
            @pl.loop(16, ch)
            def _zrest(r):
                for c in range(HH // 16):
                    inf[0][r, pl.ds(c * 16, 16)] = zv

            @pl.loop(0, tail)
            def _zrt(r):
                for c in range(HH // 16):
                    inf[1][r, pl.ds(c * 16, 16)] = zv

        plsc.subcore_barrier()

        base = wid * epw

        def issue(off, n, ivb, rbb, semi, semr):
            ci = pltpu.async_copy(i_hbm.at[pl.ds(off, n)], ivb, semi)
            cr = pltpu.async_copy(rows_hbm.at[pl.ds(off, n)], rbb, semr)
            return ci, cr

        def scat(n, ivb, rbb, ibuf, ci, cr):
            ci.wait()
            cr.wait()
            if inflate:
                @pl.loop(0, n)
                def _inf(r):
                    ibuf[r, pl.ds(0, src_width)] = rbb[r, pl.ds(0, src_width)]

                pltpu.sync_copy(ibuf, acc_sh.at[ivb], add=True)
            else:
                pltpu.sync_copy(rbb, acc_sh.at[ivb], add=True)

        ib0 = inf[0] if inflate else None
        ib1 = inf[0] if inflate else None

        @pl.loop(0, npair)
        def _pair(p):
            off0 = base + (2 * p) * ch
            h0 = issue(off0, ch, iv0, rb0, semi0, semr0)
            h1 = issue(off0 + ch, ch, iv1, rb1, semi1, semr1)
            scat(ch, iv0, rb0, ib0, *h0)
            scat(ch, iv1, rb1, ib1, *h1)

        if nodd:
            offo = base + (npair * 2) * ch
            ho = issue(offo, ch, iv0, rb0, semi0, semr0)
            scat(ch, iv0, rb0, ib0, *ho)

        if tail:
            offt = base + nfull * ch
            ht = issue(offt, tail, ivt, rbt, semi1, semr1)
            scat(tail, ivt, rbt, inf[1] if inflate else None, *ht)

        plsc.subcore_barrier()

        @pl.loop(0, _NROUND)
        def _flush(k):
            off = (k * NS + sid) * 16
            pltpu.sync_copy(acc_sh.at[pl.ds(off, 16)],
                            acc_hbm.at[cid, pl.ds(off, 16)])

        @pl.when(sid < _NXTRA)
        def _():
            off = (_NROUND * NS + sid) * 16
            pltpu.sync_copy(acc_sh.at[pl.ds(off, 16)],
                            acc_hbm.at[cid, pl.ds(off, 16)])

    return scatter_kernel


_BN_NODE = 2000


def _node1_body(h_ref, aggp_ref, na_ref, w1_ref, w2_ref, wc_ref, bh1_ref,
                gh_ref, bh_ref, hh_ref, ab2_ref, acc_ref):
    pid = pl.program_id(0)

    @pl.when(pid == 0)
    def _():
        acc_ref[...] = jnp.zeros_like(acc_ref)

    agg = aggp_ref[0]
    for k in range(1, NPART):
        agg = agg + aggp_ref[k]
    na = na_ref[...]
    hh = (jnp.dot(h_ref[...], w1_ref[...], preferred_element_type=_F32,
                  precision=_HI)
          + jnp.dot(agg, w2_ref[...], preferred_element_type=_F32,
                    precision=_HI)
          + na[:, 0:1] * wc_ref[0:1, :] + na[:, 1:2] * wc_ref[1:2, :]
          + bh1_ref[...])
    hh_ref[...] = hh
    acc_ref[0:1, :] += jnp.sum(hh, axis=0, keepdims=True)
    acc_ref[1:2, :] += jnp.sum(hh * hh, axis=0, keepdims=True)

    @pl.when(pid == pl.num_programs(0) - 1)
    def _():
        mu = acc_ref[0:1, :] * (1.0 / N)
        var = acc_ref[1:2, :] * (1.0 / N) - mu * mu
        a = gh_ref[...] * lax.rsqrt(var + 1e-5)
        b = bh_ref[...] - mu * a
        ab2_ref[...] = jnp.concatenate([a, b], axis=0)


def _node1(h, aggp, node_attr, w1, w2, wc, bh1, gh, bh):
    nblk = N // _BN_NODE
    return pl.pallas_call(
        _node1_body,
        grid=(nblk,),
        in_specs=[
            pl.BlockSpec((_BN_NODE, D), lambda n: (n, 0)),
            pl.BlockSpec((NPART, _BN_NODE, HH), lambda n: (0, n, 0)),
            pl.BlockSpec((_BN_NODE, 2), lambda n: (n, 0)),
            pl.BlockSpec((D, HH), lambda n: (0, 0)),
            pl.BlockSpec((HH, HH), lambda n: (0, 0)),
            pl.BlockSpec((2, HH), lambda n: (0, 0)),
            pl.BlockSpec((1, HH), lambda n: (0, 0)),
            pl.BlockSpec((1, HH), lambda n: (0, 0)),
            pl.BlockSpec((1, HH), lambda n: (0, 0)),
        ],
        out_specs=[
            pl.BlockSpec((_BN_NODE, HH), lambda n: (n, 0)),
            pl.BlockSpec((2, HH), lambda n: (0, 0)),
        ],
        out_shape=[
            jax.ShapeDtypeStruct((N, HH), _F32),
            jax.ShapeDtypeStruct((2, HH), _F32),
        ],
        scratch_shapes=[pltpu.VMEM((2, HH), _F32)],
    )(h, aggp, node_attr, w1, w2, wc, bh1, gh, bh)


def _node2_body(hh_ref, ab2_ref, h_ref, wh2_ref, bh2_ref, x_ref, tsp_ref,
                hn_ref, xn_ref):
    hh = jnp.maximum(hh_ref[...] * ab2_ref[0:1, :] + ab2_ref[1:2, :], 0.0)
    hn_ref[...] = (h_ref[...]
                   + jnp.dot(hh, wh2_ref[...], preferred_element_type=_F32,
                             precision=_HI) + bh2_ref[...])
    ts = tsp_ref[0]
    for k in range(1, NPART):
        ts = ts + tsp_ref[k]
    cnt = jnp.maximum(ts[:, 4:5], 1.0)
    xn_ref[...] = x_ref[...] + ts[:, 0:4] / cnt


def _node2(hh, ab2, h, wh2, bh2, x, tsp):
    nblk = N // _BN_NODE
    return pl.pallas_call(
        _node2_body,
        grid=(nblk,),
        in_specs=[
            pl.BlockSpec((_BN_NODE, HH), lambda n: (n, 0)),
            pl.BlockSpec((2, HH), lambda n: (0, 0)),
            pl.BlockSpec((_BN_NODE, D), lambda n: (n, 0)),
            pl.BlockSpec((HH, D), lambda n: (0, 0)),
            pl.BlockSpec((1, D), lambda n: (0, 0)),
            pl.BlockSpec((_BN_NODE, 4), lambda n: (n, 0)),
            pl.BlockSpec((NPART, _BN_NODE, HH), lambda n: (0, n, 0)),
        ],
        out_specs=[
            pl.BlockSpec((_BN_NODE, D), lambda n: (n, 0)),
            pl.BlockSpec((_BN_NODE, 4), lambda n: (n, 0)),
        ],
        out_shape=[
            jax.ShapeDtypeStruct((N, D), _F32),
            jax.ShapeDtypeStruct((N, 4), _F32),
        ],
    )(hh, ab2, h, wh2, bh2, x, tsp)


def kernel(h, x, edges, node_attr, we1, ge, be, we2, be2, wm, bm, wh1, bh1,
           gh, bh, wh2, bh2, wx1, bx1, wx2):
    ivec = edges[0]
    jvec = edges[1]
    w1a = we1[:D]
    w1b = we1[D:2 * D]
    wn = we1[2 * D].reshape(1, HH)
    wd = we1[2 * D + 1].reshape(1, HH)

    rptab, rqtab = _prep(h, x, w1a, w1b)

    ivs = [lax.slice_in_dim(ivec, k * ECH, (k + 1) * ECH)
           for k in range(NCHUNK)]
    jvs = [lax.slice_in_dim(jvec, k * ECH, (k + 1) * ECH)
           for k in range(NCHUNK)]

    ss = [_sc_gather(rptab, rqtab, ivs[k], jvs[k]) for k in range(NCHUNK)]
    parts = [_edge_stats(ss[k][0], ss[k][1], wn, wd) for k in range(NCHUNK)]
    ab = _stats_combine(parts, ge.reshape(1, HH), be.reshape(1, HH))

    ms, t16s = [], []
    for k in range(NCHUNK):
        mk, tk = _edge_mlp(ss[k][0], ss[k][1], wn, wd, ab, we2,
                           be2.reshape(1, HH), wm.reshape(1, HH),
                           bm.reshape(1, 1), wx1, bx1.reshape(1, HH),
                           wx2.reshape(1, HH))
        ms.append(mk)
        t16s.append(tk)

    aggps = [_sc_scatter(HH, ECH)(ms[k], ivs[k]) for k in range(NCHUNK)]
    tsps = [_sc_scatter(XW, ECH, ch=64)(t16s[k], ivs[k])
            for k in range(NCHUNK)]

    aggp = jnp.concatenate(aggps, axis=0)
    tsp = jnp.concatenate(tsps, axis=0)
    m = jnp.concatenate(ms, axis=0)

    hhpre, ab2 = _node1(h, aggp, node_attr, wh1[:D], wh1[D:2 * D],
                        wh1[2 * D:], bh1.reshape(1, HH), gh.reshape(1, HH),
                        bh.reshape(1, HH))
    h_new, x_new = _node2(hhpre, ab2, h, wh2, bh2.reshape(1, D), x, tsp)
    return (h_new, x_new, m)
